# Initial kernel scaffold; baseline (speedup 1.0000x reference)
import jax
import jax.numpy as jnp
from jax.experimental import pallas as pl


def kernel(iid, edge_index, dis, agg_src, agg_dst, pid, tid, targets, emb, pos_emb, dis_emb1, dis_emb2, tgt_emb, W_pi1, W_pi2, W_q):
    raise NotImplementedError("write your pallas kernel here")


# speedup vs baseline: 2.0955x; 2.0955x over previous
import functools
import jax
import jax.numpy as jnp
from jax import lax
from jax.experimental import pallas as pl

NUM_NODE = 100000
DIM = 128
N_ITEMS = 10000
N_EDGES = 320000
N_AGG = 10000
N_TARGETS = 512

VTILE = 2000


def _l2norm(x):
    n = jnp.sqrt(jnp.sum(x * x, axis=-1, keepdims=True))
    return x / jnp.maximum(n, 1e-12)


def _edge_softmax_agg(h, hd, src, dst, W_pi):
    ef = h[src] * h[dst] * hd
    e = (ef @ W_pi)[:, 0]
    p = jnp.exp(e)
    s = jax.ops.segment_sum(p, dst, num_segments=N_ITEMS)
    acc = jax.ops.segment_sum(h[src] * p[:, None], dst, num_segments=N_ITEMS)
    return acc / jnp.maximum(s, 1e-30)[:, None]


NK1 = 50
KT1 = NUM_NODE // NK1
KG = 5
KSUB = 8
KT2 = NUM_NODE // (KG * KSUB)
RBLK = 32
NR = N_TARGETS // RBLK


def _sumexp_body(sr_ref, emb_ref, out_ref):
    b = emb_ref[0]
    n2 = jnp.sum(b * b, axis=1)
    inv = 1.0 / jnp.maximum(jnp.sqrt(n2), 1e-12)
    logits = lax.dot_general(sr_ref[...], b, (((1,), (1,)), ((), ())),
                             preferred_element_type=jnp.float32)
    logits = 12.0 * logits * inv[None, :]
    out_ref[0] = jnp.sum(jnp.exp(logits), axis=1, keepdims=True)


def _score_body(sr_ref, lse_ref, emb_ref, out_ref):
    sr = sr_ref[...]
    lse = lse_ref[...]
    for j in range(KSUB):
        b = emb_ref[0, j]
        n2 = jnp.sum(b * b, axis=1)
        inv = 1.0 / jnp.maximum(jnp.sqrt(n2), 1e-12)
        logits = lax.dot_general(sr, b, (((1,), (1,)), ((), ())),
                                 preferred_element_type=jnp.float32)
        out_ref[:, j, :] = 12.0 * logits * inv[None, :] - lse


def _final_score(sr_n, emb):
    emb1 = emb.reshape(NK1, KT1, DIM)
    sumexp = pl.pallas_call(
        _sumexp_body,
        grid=(NK1,),
        in_specs=[
            pl.BlockSpec((N_TARGETS, DIM), lambda k: (0, 0)),
            pl.BlockSpec((1, KT1, DIM), lambda k: (k, 0, 0)),
        ],
        out_specs=pl.BlockSpec((1, N_TARGETS, 1), lambda k: (k, 0, 0)),
        out_shape=jax.ShapeDtypeStruct((NK1, N_TARGETS, 1), jnp.float32),
    )(sr_n, emb1)
    lse = jnp.log(jnp.sum(sumexp, axis=0))
    emb2 = emb.reshape(KG, KSUB, KT2, DIM)
    score = pl.pallas_call(
        _score_body,
        grid=(KG, NR),
        in_specs=[
            pl.BlockSpec((RBLK, DIM), lambda k, r: (r, 0)),
            pl.BlockSpec((RBLK, 1), lambda k, r: (r, 0)),
            pl.BlockSpec((1, KSUB, KT2, DIM), lambda k, r: (k, 0, 0, 0)),
        ],
        out_specs=pl.BlockSpec((RBLK, KSUB, KT2), lambda k, r: (r, k, 0)),
        out_shape=jax.ShapeDtypeStruct((N_TARGETS, KG * KSUB, KT2),
                                       jnp.float32),
    )(sr_n, lse, emb2)
    return score.reshape(N_TARGETS, NUM_NODE)


def kernel(iid, edge_index, dis, agg_src, agg_dst, pid, tid, targets, emb,
           pos_emb, dis_emb1, dis_emb2, tgt_emb, W_pi1, W_pi2, W_q):
    h_v = _l2norm(jnp.take(emb, iid, axis=0))
    src = edge_index[0]
    dst = edge_index[1]
    h_d1 = jnp.take(dis_emb1, dis, axis=0)
    h_d2 = jnp.take(dis_emb2, dis, axis=0)
    h1 = h_v + _edge_softmax_agg(h_v, h_d1, src, dst, W_pi1)
    h2 = h_v + _edge_softmax_agg(h_v, h_d2, dst, src, W_pi2)
    h = h1 + h2

    h_t = jnp.take(tgt_emb, tid, axis=0)
    ft_e = jnp.take(h, agg_src, axis=0)
    h_p = jnp.take(pos_emb, pid, axis=0)
    e = jnp.tanh(jnp.concatenate([ft_e, h_p], axis=-1) @ W_q)
    m = ft_e * jnp.sum(e * jnp.take(h_t, agg_dst, axis=0), axis=-1,
                       keepdims=True)
    sr = jax.ops.segment_sum(m, agg_dst, num_segments=N_TARGETS)
    sr_n = _l2norm(sr)
    return _final_score(sr_n, emb)


# speedup vs baseline: 3.8309x; 1.8282x over previous
import functools
import jax
import jax.numpy as jnp
from jax import lax
from jax.experimental import pallas as pl
from jax.experimental.pallas import tpu as pltpu
from jax.experimental.pallas import tpu_sc as plsc

NUM_NODE = 100000
DIM = 128
N_ITEMS = 10000
N_EDGES = 320000
N_AGG = 10000
N_TARGETS = 512

VTILE = 2000


def _l2norm(x):
    n = jnp.sqrt(jnp.sum(x * x, axis=-1, keepdims=True))
    return x / jnp.maximum(n, 1e-12)


NW = 32
EW = 10112
EPAD = NW * EW
ECHUNK = 64
NCHUNK = EW // ECHUNK
NPAD = 10240
RPT = NPAD // 16
JL = DIM // 16


def _edge_sweep(cid, sid, h_ref, src_ref, dst_ref, dis_ref, c_ref, out_ref,
                s_out_ref, srcb, dstb, disb, hsb, htb, cb, sacc, acc_sh,
                sem, mul_is_src):
    wid = cid * 16 + sid
    ebase = wid * EW
    iota = lax.iota(jnp.int32, 16)
    zeros16 = jnp.zeros((16,), jnp.float32)

    def _zrow(r, _):
        for j in range(JL):
            hsb[r, pl.ds(16 * j, 16)] = zeros16
        return 0
    lax.fori_loop(0, ECHUNK, _zrow, 0)

    def _zs(i, _):
        sacc[pl.ds(16 * i, 16)] = zeros16
        return 0
    lax.fori_loop(0, RPT, _zs, 0)

    abase = sid * RPT
    for k in range(RPT // ECHUNK):
        pltpu.sync_copy(hsb, acc_sh.at[pl.ds(abase + ECHUNK * k, ECHUNK)])
    plsc.subcore_barrier()

    mul_ref = hsb if mul_is_src else htb
    idx_ref = dstb if mul_is_src else srcb

    def _chunk(g, _):
        off = ebase + g * ECHUNK
        pltpu.sync_copy(src_ref.at[pl.ds(off, ECHUNK)], srcb)
        pltpu.sync_copy(dst_ref.at[pl.ds(off, ECHUNK)], dstb)
        pltpu.sync_copy(dis_ref.at[pl.ds(off, ECHUNK)], disb)
        ga = pltpu.async_copy(h_ref.at[srcb], hsb, sem)
        gb = pltpu.async_copy(h_ref.at[dstb], htb, sem)
        gc = pltpu.async_copy(c_ref.at[disb], cb, sem)
        ga.wait()
        gb.wait()
        gc.wait()

        def _group(gi, _):
            tvec = idx_ref[pl.ds(16 * gi, 16)]
            for ei in range(16):
                i = 16 * gi + ei
                acc_v = zeros16
                for j in range(JL):
                    a = hsb[i, pl.ds(16 * j, 16)]
                    b = htb[i, pl.ds(16 * j, 16)]
                    c = cb[i, pl.ds(16 * j, 16)]
                    acc_v = acc_v + a * b * c
                e = jnp.sum(acc_v)
                p_v = jnp.exp(lax.broadcast_in_dim(e, (16,), ()))
                for j in range(JL):
                    mul_ref[i, pl.ds(16 * j, 16)] = (
                        p_v * mul_ref[i, pl.ds(16 * j, 16)])
                t_i = tvec[ei]
                off16 = pl.multiple_of((t_i // 16) * 16, 16)
                lane = lax.broadcast_in_dim(t_i - off16, (16,), ())
                sval = sacc[pl.ds(off16, 16)]
                sacc[pl.ds(off16, 16)] = jnp.where(
                    iota == lane, sval + p_v, sval)
            return 0

        lax.fori_loop(0, ECHUNK // 16, _group, 0)
        pltpu.sync_copy(mul_ref, acc_sh.at[idx_ref], add=True)
        return 0

    lax.fori_loop(0, NCHUNK, _chunk, 0)
    plsc.subcore_barrier()
    pltpu.sync_copy(acc_sh.at[pl.ds(abase, RPT)],
                    out_ref.at[cid, pl.ds(abase, RPT)])
    pltpu.sync_copy(sacc, s_out_ref.at[wid])
    plsc.subcore_barrier()


def _make_edge_kernel():
    mesh = plsc.VectorSubcoreMesh(core_axis_name="c", subcore_axis_name="s")
    f32 = jnp.float32

    @functools.partial(
        pl.kernel,
        out_type=[
            jax.ShapeDtypeStruct((2, NPAD, DIM), f32),
            jax.ShapeDtypeStruct((2, NPAD, DIM), f32),
            jax.ShapeDtypeStruct((NW, NPAD), f32),
            jax.ShapeDtypeStruct((NW, NPAD), f32),
        ],
        mesh=mesh,
        compiler_params=pltpu.CompilerParams(needs_layout_passes=False),
        scratch_types=[
            pltpu.VMEM((ECHUNK,), jnp.int32),
            pltpu.VMEM((ECHUNK,), jnp.int32),
            pltpu.VMEM((ECHUNK,), jnp.int32),
            pltpu.VMEM((ECHUNK, DIM), f32),
            pltpu.VMEM((ECHUNK, DIM), f32),
            pltpu.VMEM((ECHUNK, DIM), f32),
            pltpu.VMEM((NPAD,), f32),
            pltpu.VMEM_SHARED((NPAD, DIM), f32),
            pltpu.SemaphoreType.DMA,
        ],
    )
    def k(h_ref, src_ref, dst_ref, dis_ref, c1_ref, c2_ref,
          out1_ref, out2_ref, s1_ref, s2_ref,
          srcb, dstb, disb, hsb, htb, cb, sacc, acc_sh, sem):
        cid = lax.axis_index("c")
        sid = lax.axis_index("s")
        _edge_sweep(cid, sid, h_ref, src_ref, dst_ref, dis_ref, c1_ref,
                    out1_ref, s1_ref, srcb, dstb, disb, hsb, htb, cb,
                    sacc, acc_sh, sem, mul_is_src=True)
        _edge_sweep(cid, sid, h_ref, src_ref, dst_ref, dis_ref, c2_ref,
                    out2_ref, s2_ref, srcb, dstb, disb, hsb, htb, cb,
                    sacc, acc_sh, sem, mul_is_src=False)

    return k


def _gat_both_layers(h_v, src, dst, dis, c1, c2):
    npad_e = EPAD - N_EDGES
    srcp = jnp.concatenate([src, jnp.full((npad_e,), N_ITEMS, jnp.int32)])
    dstp = jnp.concatenate([dst, jnp.full((npad_e,), N_ITEMS, jnp.int32)])
    disp = jnp.concatenate([dis, jnp.zeros((npad_e,), jnp.int32)])
    h_pad = jnp.concatenate([h_v, jnp.zeros((NPAD - N_ITEMS, DIM),
                                            jnp.float32)])
    out1, out2, s1p, s2p = _make_edge_kernel()(
        h_pad, srcp, dstp, disp, c1, c2)
    acc1 = (out1[0] + out1[1])[:N_ITEMS]
    acc2 = (out2[0] + out2[1])[:N_ITEMS]
    s1 = jnp.sum(s1p, axis=0)[:N_ITEMS]
    s2 = jnp.sum(s2p, axis=0)[:N_ITEMS]
    a1 = acc1 / jnp.maximum(s1, 1e-30)[:, None]
    a2 = acc2 / jnp.maximum(s2, 1e-30)[:, None]
    return a1, a2


NK1 = 50
KT1 = NUM_NODE // NK1
KG = 5
KSUB = 8
KT2 = NUM_NODE // (KG * KSUB)
RBLK = 32
NR = N_TARGETS // RBLK


def _sumexp_body(sr_ref, emb_ref, out_ref):
    b = emb_ref[0]
    n2 = jnp.sum(b * b, axis=1)
    inv = 1.0 / jnp.maximum(jnp.sqrt(n2), 1e-12)
    logits = lax.dot_general(sr_ref[...], b, (((1,), (1,)), ((), ())),
                             preferred_element_type=jnp.float32)
    logits = 12.0 * logits * inv[None, :]
    out_ref[0] = jnp.sum(jnp.exp(logits), axis=1, keepdims=True)


def _score_body(sr_ref, lse_ref, emb_ref, out_ref):
    sr = sr_ref[...]
    lse = lse_ref[...]
    for j in range(KSUB):
        b = emb_ref[0, j]
        n2 = jnp.sum(b * b, axis=1)
        inv = 1.0 / jnp.maximum(jnp.sqrt(n2), 1e-12)
        logits = lax.dot_general(sr, b, (((1,), (1,)), ((), ())),
                                 preferred_element_type=jnp.float32)
        out_ref[:, j, :] = 12.0 * logits * inv[None, :] - lse


def _final_score(sr_n, emb):
    emb1 = emb.reshape(NK1, KT1, DIM)
    sumexp = pl.pallas_call(
        _sumexp_body,
        grid=(NK1,),
        in_specs=[
            pl.BlockSpec((N_TARGETS, DIM), lambda k: (0, 0)),
            pl.BlockSpec((1, KT1, DIM), lambda k: (k, 0, 0)),
        ],
        out_specs=pl.BlockSpec((1, N_TARGETS, 1), lambda k: (k, 0, 0)),
        out_shape=jax.ShapeDtypeStruct((NK1, N_TARGETS, 1), jnp.float32),
    )(sr_n, emb1)
    lse = jnp.log(jnp.sum(sumexp, axis=0))
    emb2 = emb.reshape(KG, KSUB, KT2, DIM)
    score = pl.pallas_call(
        _score_body,
        grid=(KG, NR),
        in_specs=[
            pl.BlockSpec((RBLK, DIM), lambda k, r: (r, 0)),
            pl.BlockSpec((RBLK, 1), lambda k, r: (r, 0)),
            pl.BlockSpec((1, KSUB, KT2, DIM), lambda k, r: (k, 0, 0, 0)),
        ],
        out_specs=pl.BlockSpec((RBLK, KSUB, KT2), lambda k, r: (r, k, 0)),
        out_shape=jax.ShapeDtypeStruct((N_TARGETS, KG * KSUB, KT2),
                                       jnp.float32),
    )(sr_n, lse, emb2)
    return score.reshape(N_TARGETS, NUM_NODE)


def kernel(iid, edge_index, dis, agg_src, agg_dst, pid, tid, targets, emb,
           pos_emb, dis_emb1, dis_emb2, tgt_emb, W_pi1, W_pi2, W_q):
    h_v = _l2norm(jnp.take(emb, iid, axis=0))
    src = edge_index[0]
    dst = edge_index[1]
    c1 = dis_emb1 * W_pi1[:, 0][None, :]
    c2 = dis_emb2 * W_pi2[:, 0][None, :]
    a1, a2 = _gat_both_layers(h_v, src, dst, dis, c1, c2)
    h = 2.0 * h_v + a1 + a2

    h_t = jnp.take(tgt_emb, tid, axis=0)
    ft_e = jnp.take(h, agg_src, axis=0)
    h_p = jnp.take(pos_emb, pid, axis=0)
    e = jnp.tanh(jnp.concatenate([ft_e, h_p], axis=-1) @ W_q)
    m = ft_e * jnp.sum(e * jnp.take(h_t, agg_dst, axis=0), axis=-1,
                       keepdims=True)
    sr = jax.ops.segment_sum(m, agg_dst, num_segments=N_TARGETS)
    sr_n = _l2norm(sr)
    return _final_score(sr_n, emb)


# speedup vs baseline: 4.3209x; 1.1279x over previous
import functools
import jax
import jax.numpy as jnp
from jax import lax
from jax.experimental import pallas as pl
from jax.experimental.pallas import tpu as pltpu
from jax.experimental.pallas import tpu_sc as plsc

NUM_NODE = 100000
DIM = 128
N_ITEMS = 10000
N_EDGES = 320000
N_AGG = 10000
N_TARGETS = 512

VTILE = 2000


def _l2norm(x):
    n = jnp.sqrt(jnp.sum(x * x, axis=-1, keepdims=True))
    return x / jnp.maximum(n, 1e-12)


NW = 32
EW = N_EDGES // NW
ECHUNK = 32
NCHUNK = 312
EREM = EW - NCHUNK * ECHUNK
NPAD = 10240
RPT = NPAD // 16
JL = DIM // 16


def _edge_sweep(cid, sid, h_ref, src_ref, dst_ref, dis_ref, c_ref, out_ref,
                s_out_ref, bufs, rbufs, sacc, acc_sh, sems, mul_is_src):
    wid = cid * 16 + sid
    ebase = wid * EW
    iota = lax.iota(jnp.int32, 16)
    zeros16 = jnp.zeros((16,), jnp.float32)
    hsb0 = bufs[0][3]

    def _zrow(r, _):
        for j in range(JL):
            hsb0[r, pl.ds(16 * j, 16)] = zeros16
        return 0
    lax.fori_loop(0, ECHUNK, _zrow, 0)

    def _zs(i, _):
        sacc[pl.ds(16 * i, 16)] = zeros16
        return 0
    lax.fori_loop(0, RPT, _zs, 0)

    abase = sid * RPT
    for k in range(RPT // ECHUNK):
        pltpu.sync_copy(hsb0, acc_sh.at[pl.ds(abase + ECHUNK * k, ECHUNK)])
    plsc.subcore_barrier()

    def _fetch(g, par):
        srcb, dstb, disb, hsb, htb, cb = bufs[par]
        off = ebase + g * ECHUNK
        pltpu.sync_copy(src_ref.at[pl.ds(off, ECHUNK)], srcb)
        pltpu.sync_copy(dst_ref.at[pl.ds(off, ECHUNK)], dstb)
        pltpu.sync_copy(dis_ref.at[pl.ds(off, ECHUNK)], disb)
        pltpu.async_copy(h_ref.at[srcb], hsb, sems[par])
        pltpu.async_copy(h_ref.at[dstb], htb, sems[par])
        pltpu.async_copy(c_ref.at[disb], cb, sems[par])

    def _do_group(gi, tup):
        srcb, dstb, disb, hsb, htb, cb = tup
        mul_ref = hsb if mul_is_src else htb
        idx_ref = dstb if mul_is_src else srcb
        tvec = idx_ref[pl.ds(16 * gi, 16)]
        for ei in range(16):
            i = 16 * gi + ei
            acc_v = zeros16
            for j in range(JL):
                a = hsb[i, pl.ds(16 * j, 16)]
                b = htb[i, pl.ds(16 * j, 16)]
                c = cb[i, pl.ds(16 * j, 16)]
                acc_v = acc_v + a * b * c
            e = jnp.sum(acc_v)
            p_v = jnp.exp(lax.broadcast_in_dim(e, (16,), ()))
            for j in range(JL):
                mul_ref[i, pl.ds(16 * j, 16)] = (
                    p_v * mul_ref[i, pl.ds(16 * j, 16)])
            t_i = tvec[ei]
            off16 = pl.multiple_of((t_i // 16) * 16, 16)
            lane = lax.broadcast_in_dim(t_i - off16, (16,), ())
            sval = sacc[pl.ds(off16, 16)]
            sacc[pl.ds(off16, 16)] = jnp.where(
                iota == lane, sval + p_v, sval)

    def _consume(par):
        tup = bufs[par]
        srcb, dstb, disb, hsb, htb, cb = tup
        mul_ref = hsb if mul_is_src else htb
        idx_ref = dstb if mul_is_src else srcb
        for dst_buf in (hsb, htb, cb):
            pltpu.make_async_copy(h_ref.at[srcb], dst_buf, sems[par]).wait()
        lax.fori_loop(0, ECHUNK // 16,
                      lambda gi, _: (_do_group(gi, tup), 0)[1], 0)
        pltpu.sync_copy(mul_ref, acc_sh.at[idx_ref], add=True)

    _fetch(0, 0)
    _fetch(1, 1)

    def _pipe(gg, _):
        for par in range(2):
            g = 2 * gg + par
            _consume(par)

            @pl.when(g + 2 < NCHUNK)
            def _():
                _fetch(g + 2, par)
        return 0

    lax.fori_loop(0, NCHUNK // 2, _pipe, 0)

    rsrcb, rdstb, rdisb, rhsb, rhtb, rcb = rbufs
    roff = ebase + NCHUNK * ECHUNK
    pltpu.sync_copy(src_ref.at[pl.ds(roff, EREM)], rsrcb)
    pltpu.sync_copy(dst_ref.at[pl.ds(roff, EREM)], rdstb)
    pltpu.sync_copy(dis_ref.at[pl.ds(roff, EREM)], rdisb)
    pltpu.async_copy(h_ref.at[rsrcb], rhsb, sems[0])
    pltpu.async_copy(h_ref.at[rdstb], rhtb, sems[0])
    pltpu.async_copy(c_ref.at[rdisb], rcb, sems[0])
    for dst_buf in (rhsb, rhtb, rcb):
        pltpu.make_async_copy(h_ref.at[rsrcb], dst_buf, sems[0]).wait()
    _do_group(0, rbufs)
    rmul = rhsb if mul_is_src else rhtb
    ridx = rdstb if mul_is_src else rsrcb
    pltpu.sync_copy(rmul, acc_sh.at[ridx], add=True)

    plsc.subcore_barrier()
    pltpu.sync_copy(acc_sh.at[pl.ds(abase, RPT)],
                    out_ref.at[cid, pl.ds(abase, RPT)])
    pltpu.sync_copy(sacc, s_out_ref.at[wid])
    plsc.subcore_barrier()


def _make_edge_kernel():
    mesh = plsc.VectorSubcoreMesh(core_axis_name="c", subcore_axis_name="s")
    f32 = jnp.float32

    @functools.partial(
        pl.kernel,
        out_type=[
            jax.ShapeDtypeStruct((2, NPAD, DIM), f32),
            jax.ShapeDtypeStruct((2, NPAD, DIM), f32),
            jax.ShapeDtypeStruct((NW, NPAD), f32),
            jax.ShapeDtypeStruct((NW, NPAD), f32),
        ],
        mesh=mesh,
        compiler_params=pltpu.CompilerParams(needs_layout_passes=False),
        scratch_types=(
            [pltpu.VMEM((ECHUNK,), jnp.int32)] * 3
            + [pltpu.VMEM((ECHUNK, DIM), f32)] * 3
            + [pltpu.VMEM((ECHUNK,), jnp.int32)] * 3
            + [pltpu.VMEM((ECHUNK, DIM), f32)] * 3
            + [pltpu.VMEM((EREM,), jnp.int32)] * 3
            + [pltpu.VMEM((EREM, DIM), f32)] * 3
            + [
                pltpu.VMEM((NPAD,), f32),
                pltpu.VMEM_SHARED((NPAD, DIM), f32),
                pltpu.SemaphoreType.DMA,
                pltpu.SemaphoreType.DMA,
            ]
        ),
    )
    def k(h_ref, src_ref, dst_ref, dis_ref, c1_ref, c2_ref,
          out1_ref, out2_ref, s1_ref, s2_ref,
          s0, d0, i0, hs0, ht0, c0, s1b, d1, i1, hs1, ht1, c1b,
          sr, dr, ir, hsr, htr, cr, sacc, acc_sh, sem0, sem1):
        cid = lax.axis_index("c")
        sid = lax.axis_index("s")
        bufs = ((s0, d0, i0, hs0, ht0, c0), (s1b, d1, i1, hs1, ht1, c1b))
        rbufs = (sr, dr, ir, hsr, htr, cr)
        sems = (sem0, sem1)
        _edge_sweep(cid, sid, h_ref, src_ref, dst_ref, dis_ref, c1_ref,
                    out1_ref, s1_ref, bufs, rbufs, sacc, acc_sh, sems,
                    mul_is_src=True)
        _edge_sweep(cid, sid, h_ref, src_ref, dst_ref, dis_ref, c2_ref,
                    out2_ref, s2_ref, bufs, rbufs, sacc, acc_sh, sems,
                    mul_is_src=False)

    return k


def _gat_both_layers(h_v, src, dst, dis, c1, c2):
    out1, out2, s1p, s2p = _make_edge_kernel()(
        h_v, src, dst, dis, c1, c2)
    acc1 = (out1[0] + out1[1])[:N_ITEMS]
    acc2 = (out2[0] + out2[1])[:N_ITEMS]
    s1 = jnp.sum(s1p, axis=0)[:N_ITEMS]
    s2 = jnp.sum(s2p, axis=0)[:N_ITEMS]
    a1 = acc1 / jnp.maximum(s1, 1e-30)[:, None]
    a2 = acc2 / jnp.maximum(s2, 1e-30)[:, None]
    return a1, a2


NK1 = 50
KT1 = NUM_NODE // NK1
KG = 5
KSUB = 8
KT2 = NUM_NODE // (KG * KSUB)
RBLK = 32
NR = N_TARGETS // RBLK


def _sumexp_body(sr_ref, emb_ref, out_ref):
    b = emb_ref[0]
    n2 = jnp.sum(b * b, axis=1)
    inv = 1.0 / jnp.maximum(jnp.sqrt(n2), 1e-12)
    logits = lax.dot_general(sr_ref[...], b, (((1,), (1,)), ((), ())),
                             preferred_element_type=jnp.float32)
    logits = 12.0 * logits * inv[None, :]
    out_ref[0] = jnp.sum(jnp.exp(logits), axis=1, keepdims=True)


def _score_body(sr_ref, lse_ref, emb_ref, out_ref):
    sr = sr_ref[...]
    lse = lse_ref[...]
    for j in range(KSUB):
        b = emb_ref[0, j]
        n2 = jnp.sum(b * b, axis=1)
        inv = 1.0 / jnp.maximum(jnp.sqrt(n2), 1e-12)
        logits = lax.dot_general(sr, b, (((1,), (1,)), ((), ())),
                                 preferred_element_type=jnp.float32)
        out_ref[:, j, :] = 12.0 * logits * inv[None, :] - lse


def _final_score(sr_n, emb):
    emb1 = emb.reshape(NK1, KT1, DIM)
    sumexp = pl.pallas_call(
        _sumexp_body,
        grid=(NK1,),
        in_specs=[
            pl.BlockSpec((N_TARGETS, DIM), lambda k: (0, 0)),
            pl.BlockSpec((1, KT1, DIM), lambda k: (k, 0, 0)),
        ],
        out_specs=pl.BlockSpec((1, N_TARGETS, 1), lambda k: (k, 0, 0)),
        out_shape=jax.ShapeDtypeStruct((NK1, N_TARGETS, 1), jnp.float32),
    )(sr_n, emb1)
    lse = jnp.log(jnp.sum(sumexp, axis=0))
    emb2 = emb.reshape(KG, KSUB, KT2, DIM)
    score = pl.pallas_call(
        _score_body,
        grid=(KG, NR),
        in_specs=[
            pl.BlockSpec((RBLK, DIM), lambda k, r: (r, 0)),
            pl.BlockSpec((RBLK, 1), lambda k, r: (r, 0)),
            pl.BlockSpec((1, KSUB, KT2, DIM), lambda k, r: (k, 0, 0, 0)),
        ],
        out_specs=pl.BlockSpec((RBLK, KSUB, KT2), lambda k, r: (r, k, 0)),
        out_shape=jax.ShapeDtypeStruct((N_TARGETS, KG * KSUB, KT2),
                                       jnp.float32),
    )(sr_n, lse, emb2)
    return score.reshape(N_TARGETS, NUM_NODE)


def kernel(iid, edge_index, dis, agg_src, agg_dst, pid, tid, targets, emb,
           pos_emb, dis_emb1, dis_emb2, tgt_emb, W_pi1, W_pi2, W_q):
    h_v = _l2norm(jnp.take(emb, iid, axis=0))
    src = edge_index[0]
    dst = edge_index[1]
    c1 = dis_emb1 * W_pi1[:, 0][None, :]
    c2 = dis_emb2 * W_pi2[:, 0][None, :]
    a1, a2 = _gat_both_layers(h_v, src, dst, dis, c1, c2)
    h = 2.0 * h_v + a1 + a2

    h_t = jnp.take(tgt_emb, tid, axis=0)
    ft_e = jnp.take(h, agg_src, axis=0)
    h_p = jnp.take(pos_emb, pid, axis=0)
    e = jnp.tanh(jnp.concatenate([ft_e, h_p], axis=-1) @ W_q)
    m = ft_e * jnp.sum(e * jnp.take(h_t, agg_dst, axis=0), axis=-1,
                       keepdims=True)
    sr = jax.ops.segment_sum(m, agg_dst, num_segments=N_TARGETS)
    sr_n = _l2norm(sr)
    return _final_score(sr_n, emb)


# speedup vs baseline: 4.7866x; 1.1078x over previous
import functools
import jax
import jax.numpy as jnp
from jax import lax
from jax.experimental import pallas as pl
from jax.experimental.pallas import tpu as pltpu
from jax.experimental.pallas import tpu_sc as plsc

NUM_NODE = 100000
DIM = 128
N_ITEMS = 10000
N_EDGES = 320000
N_AGG = 10000
N_TARGETS = 512

VTILE = 2000


def _l2norm(x):
    n = jnp.sqrt(jnp.sum(x * x, axis=-1, keepdims=True))
    return x / jnp.maximum(n, 1e-12)


NW = 32
EW = N_EDGES // NW
ECHUNK = 32
NCHUNK = 312
EREM = EW - NCHUNK * ECHUNK
NPAD = 10240
RPT = NPAD // 16
JL = DIM // 16


def _edge_sweep(cid, sid, h_ref, ipk_ref, src_ref, dst_ref, dis_ref, c_ref,
                out_ref, s_out_ref, bufs, rbufs, sacc, acc_sh, sems,
                mul_is_src):
    wid = cid * 16 + sid
    ebase = wid * EW
    iota = lax.iota(jnp.int32, 16)
    zeros16 = jnp.zeros((16,), jnp.float32)
    hsb0 = bufs[0][1]

    def _zrow(r, _):
        for j in range(JL):
            hsb0[r, pl.ds(16 * j, 16)] = zeros16
        return 0
    lax.fori_loop(0, ECHUNK, _zrow, 0)

    def _zs(i, _):
        sacc[pl.ds(16 * i, 16)] = zeros16
        return 0
    lax.fori_loop(0, RPT, _zs, 0)

    abase = sid * RPT
    for k in range(RPT // ECHUNK):
        pltpu.sync_copy(hsb0, acc_sh.at[pl.ds(abase + ECHUNK * k, ECHUNK)])
    plsc.subcore_barrier()

    def _fetch(g, par):
        ib, hsb, htb, cb = bufs[par]
        pltpu.sync_copy(ipk_ref.at[wid, g], ib)
        pltpu.async_copy(h_ref.at[ib.at[0]], hsb, sems[par])
        pltpu.async_copy(h_ref.at[ib.at[1]], htb, sems[par])
        pltpu.async_copy(c_ref.at[ib.at[2]], cb, sems[par])

    def _do_group(gi, tup):
        ib, hsb, htb, cb = tup
        mul_ref = hsb if mul_is_src else htb
        trow = 1 if mul_is_src else 0
        tvec = ib[trow, pl.ds(16 * gi, 16)]
        for ei in range(16):
            i = 16 * gi + ei
            acc_v = zeros16
            for j in range(JL):
                a = hsb[i, pl.ds(16 * j, 16)]
                b = htb[i, pl.ds(16 * j, 16)]
                c = cb[i, pl.ds(16 * j, 16)]
                acc_v = acc_v + a * b * c
            e = jnp.sum(acc_v)
            p_v = jnp.exp(lax.broadcast_in_dim(e, (16,), ()))
            for j in range(JL):
                mul_ref[i, pl.ds(16 * j, 16)] = (
                    p_v * mul_ref[i, pl.ds(16 * j, 16)])
            t_i = tvec[ei]
            off16 = pl.multiple_of((t_i // 16) * 16, 16)
            lane = lax.broadcast_in_dim(t_i - off16, (16,), ())
            sval = sacc[pl.ds(off16, 16)]
            sacc[pl.ds(off16, 16)] = jnp.where(
                iota == lane, sval + p_v, sval)

    def _consume(par):
        tup = bufs[par]
        ib, hsb, htb, cb = tup
        mul_ref = hsb if mul_is_src else htb
        trow = 1 if mul_is_src else 0
        for dst_buf in (hsb, htb, cb):
            pltpu.make_async_copy(h_ref.at[ib.at[0]], dst_buf,
                                  sems[par]).wait()
        lax.fori_loop(0, ECHUNK // 16,
                      lambda gi, _: (_do_group(gi, tup), 0)[1], 0)
        pltpu.sync_copy(mul_ref, acc_sh.at[ib.at[trow]], add=True)

    _fetch(0, 0)
    _fetch(1, 1)

    def _pipe(gg, _):
        for par in range(2):
            g = 2 * gg + par
            _consume(par)

            @pl.when(g + 2 < NCHUNK)
            def _():
                _fetch(g + 2, par)
        return 0

    lax.fori_loop(0, NCHUNK // 2, _pipe, 0)

    rib, rhsb, rhtb, rcb = rbufs
    roff = ebase + NCHUNK * ECHUNK
    pltpu.sync_copy(src_ref.at[pl.ds(roff, EREM)], rib.at[0])
    pltpu.sync_copy(dst_ref.at[pl.ds(roff, EREM)], rib.at[1])
    pltpu.sync_copy(dis_ref.at[pl.ds(roff, EREM)], rib.at[2])
    pltpu.async_copy(h_ref.at[rib.at[0]], rhsb, sems[0])
    pltpu.async_copy(h_ref.at[rib.at[1]], rhtb, sems[0])
    pltpu.async_copy(c_ref.at[rib.at[2]], rcb, sems[0])
    for dst_buf in (rhsb, rhtb, rcb):
        pltpu.make_async_copy(h_ref.at[rib.at[0]], dst_buf, sems[0]).wait()
    _do_group(0, rbufs)
    rmul = rhsb if mul_is_src else rhtb
    rtrow = 1 if mul_is_src else 0
    pltpu.sync_copy(rmul, acc_sh.at[rib.at[rtrow]], add=True)

    plsc.subcore_barrier()
    pltpu.sync_copy(acc_sh.at[pl.ds(abase, RPT)],
                    out_ref.at[cid, pl.ds(abase, RPT)])
    pltpu.sync_copy(sacc, s_out_ref.at[wid])
    plsc.subcore_barrier()


def _make_edge_kernel():
    mesh = plsc.VectorSubcoreMesh(core_axis_name="c", subcore_axis_name="s")
    f32 = jnp.float32

    @functools.partial(
        pl.kernel,
        out_type=[
            jax.ShapeDtypeStruct((2, NPAD, DIM), f32),
            jax.ShapeDtypeStruct((2, NPAD, DIM), f32),
            jax.ShapeDtypeStruct((NW, NPAD), f32),
            jax.ShapeDtypeStruct((NW, NPAD), f32),
        ],
        mesh=mesh,
        compiler_params=pltpu.CompilerParams(needs_layout_passes=False),
        scratch_types=(
            [pltpu.VMEM((3, ECHUNK), jnp.int32)]
            + [pltpu.VMEM((ECHUNK, DIM), f32)] * 3
            + [pltpu.VMEM((3, ECHUNK), jnp.int32)]
            + [pltpu.VMEM((ECHUNK, DIM), f32)] * 3
            + [pltpu.VMEM((3, EREM), jnp.int32)]
            + [pltpu.VMEM((EREM, DIM), f32)] * 3
            + [
                pltpu.VMEM((NPAD,), f32),
                pltpu.VMEM_SHARED((NPAD, DIM), f32),
                pltpu.SemaphoreType.DMA,
                pltpu.SemaphoreType.DMA,
            ]
        ),
    )
    def k(h_ref, ipk_ref, src_ref, dst_ref, dis_ref, c1_ref, c2_ref,
          out1_ref, out2_ref, s1_ref, s2_ref,
          ib0, hs0, ht0, c0, ib1, hs1, ht1, c1b,
          ibr, hsr, htr, cr, sacc, acc_sh, sem0, sem1):
        cid = lax.axis_index("c")
        sid = lax.axis_index("s")
        bufs = ((ib0, hs0, ht0, c0), (ib1, hs1, ht1, c1b))
        rbufs = (ibr, hsr, htr, cr)
        sems = (sem0, sem1)
        _edge_sweep(cid, sid, h_ref, ipk_ref, src_ref, dst_ref, dis_ref,
                    c1_ref, out1_ref, s1_ref, bufs, rbufs, sacc, acc_sh,
                    sems, mul_is_src=True)
        _edge_sweep(cid, sid, h_ref, ipk_ref, src_ref, dst_ref, dis_ref,
                    c2_ref, out2_ref, s2_ref, bufs, rbufs, sacc, acc_sh,
                    sems, mul_is_src=False)

    return k


def _gat_both_layers(h_v, src, dst, dis, c1, c2):
    full = NCHUNK * ECHUNK
    stack = jnp.stack([src, dst, dis])
    ipk = stack.reshape(3, NW, EW)[:, :, :full].reshape(
        3, NW, NCHUNK, ECHUNK).transpose(1, 2, 0, 3)
    out1, out2, s1p, s2p = _make_edge_kernel()(
        h_v, ipk, src, dst, dis, c1, c2)
    acc1 = (out1[0] + out1[1])[:N_ITEMS]
    acc2 = (out2[0] + out2[1])[:N_ITEMS]
    s1 = jnp.sum(s1p, axis=0)[:N_ITEMS]
    s2 = jnp.sum(s2p, axis=0)[:N_ITEMS]
    a1 = acc1 / jnp.maximum(s1, 1e-30)[:, None]
    a2 = acc2 / jnp.maximum(s2, 1e-30)[:, None]
    return a1, a2


NK1 = 50
KT1 = NUM_NODE // NK1
KG = 5
KSUB = 8
KT2 = NUM_NODE // (KG * KSUB)
RBLK = 32
NR = N_TARGETS // RBLK


def _sumexp_body(sr_ref, emb_ref, out_ref):
    b = emb_ref[0]
    n2 = jnp.sum(b * b, axis=1)
    inv = 1.0 / jnp.maximum(jnp.sqrt(n2), 1e-12)
    logits = lax.dot_general(sr_ref[...], b, (((1,), (1,)), ((), ())),
                             preferred_element_type=jnp.float32)
    logits = 12.0 * logits * inv[None, :]
    out_ref[0] = jnp.sum(jnp.exp(logits), axis=1, keepdims=True)


def _score_body(sr_ref, lse_ref, emb_ref, out_ref):
    sr = sr_ref[...]
    lse = lse_ref[...]
    for j in range(KSUB):
        b = emb_ref[0, j]
        n2 = jnp.sum(b * b, axis=1)
        inv = 1.0 / jnp.maximum(jnp.sqrt(n2), 1e-12)
        logits = lax.dot_general(sr, b, (((1,), (1,)), ((), ())),
                                 preferred_element_type=jnp.float32)
        out_ref[:, j, :] = 12.0 * logits * inv[None, :] - lse


def _final_score(sr_n, emb):
    emb1 = emb.reshape(NK1, KT1, DIM)
    sumexp = pl.pallas_call(
        _sumexp_body,
        grid=(NK1,),
        in_specs=[
            pl.BlockSpec((N_TARGETS, DIM), lambda k: (0, 0)),
            pl.BlockSpec((1, KT1, DIM), lambda k: (k, 0, 0)),
        ],
        out_specs=pl.BlockSpec((1, N_TARGETS, 1), lambda k: (k, 0, 0)),
        out_shape=jax.ShapeDtypeStruct((NK1, N_TARGETS, 1), jnp.float32),
    )(sr_n, emb1)
    lse = jnp.log(jnp.sum(sumexp, axis=0))
    emb2 = emb.reshape(KG, KSUB, KT2, DIM)
    score = pl.pallas_call(
        _score_body,
        grid=(KG, NR),
        in_specs=[
            pl.BlockSpec((RBLK, DIM), lambda k, r: (r, 0)),
            pl.BlockSpec((RBLK, 1), lambda k, r: (r, 0)),
            pl.BlockSpec((1, KSUB, KT2, DIM), lambda k, r: (k, 0, 0, 0)),
        ],
        out_specs=pl.BlockSpec((RBLK, KSUB, KT2), lambda k, r: (r, k, 0)),
        out_shape=jax.ShapeDtypeStruct((N_TARGETS, KG * KSUB, KT2),
                                       jnp.float32),
    )(sr_n, lse, emb2)
    return score.reshape(N_TARGETS, NUM_NODE)


def kernel(iid, edge_index, dis, agg_src, agg_dst, pid, tid, targets, emb,
           pos_emb, dis_emb1, dis_emb2, tgt_emb, W_pi1, W_pi2, W_q):
    h_v = _l2norm(jnp.take(emb, iid, axis=0))
    src = edge_index[0]
    dst = edge_index[1]
    c1 = dis_emb1 * W_pi1[:, 0][None, :]
    c2 = dis_emb2 * W_pi2[:, 0][None, :]
    a1, a2 = _gat_both_layers(h_v, src, dst, dis, c1, c2)
    h = 2.0 * h_v + a1 + a2

    h_t = jnp.take(tgt_emb, tid, axis=0)
    ft_e = jnp.take(h, agg_src, axis=0)
    h_p = jnp.take(pos_emb, pid, axis=0)
    e = jnp.tanh(jnp.concatenate([ft_e, h_p], axis=-1) @ W_q)
    m = ft_e * jnp.sum(e * jnp.take(h_t, agg_dst, axis=0), axis=-1,
                       keepdims=True)
    sr = jax.ops.segment_sum(m, agg_dst, num_segments=N_TARGETS)
    sr_n = _l2norm(sr)
    return _final_score(sr_n, emb)
